# Initial kernel scaffold; baseline (speedup 1.0000x reference)
import jax
import jax.numpy as jnp
from jax.experimental import pallas as pl


def kernel(x, conv_w, conv_b, score_w, score_b, loc_w, loc_b, img_size):
    raise NotImplementedError("write your pallas kernel here")


# speedup vs baseline: 81.4062x; 81.4062x over previous
import jax
import jax.numpy as jnp
import numpy as np
from jax.experimental import pallas as pl

_N_IN = 12000
_N_OUT = 2000
_ROWS = 8
_PAD = 12288
_COLS = _PAD // _ROWS
_THRESH = 0.7


def _anchor_base(base_size=16, ratios=(0.5, 1, 2), scales=(8, 16, 32)):
    py = base_size / 2.0
    px = base_size / 2.0
    anchors = []
    for r in ratios:
        for s in scales:
            h = base_size * s * np.sqrt(r)
            w = base_size * s * np.sqrt(1.0 / r)
            anchors.append([py - h / 2.0, px - w / 2.0, py + h / 2.0, px + w / 2.0])
    return jnp.asarray(np.array(anchors, dtype=np.float32))


def _shifted_anchors(anchors, feat_stride, hh, ww):
    shift_y = jnp.arange(0, hh * feat_stride, feat_stride, dtype=jnp.float32)
    shift_x = jnp.arange(0, ww * feat_stride, feat_stride, dtype=jnp.float32)
    sy, sx = jnp.meshgrid(shift_y, shift_x, indexing='ij')
    shift = jnp.stack([sy.ravel(), sx.ravel(), sy.ravel(), sx.ravel()], axis=1)
    A = anchors.shape[0]
    K = shift.shape[0]
    all_anchors = anchors.reshape(1, A, 4) + shift.reshape(K, 1, 4)
    return all_anchors.reshape(K * A, 4)


def _loc2bbox(src, loc):
    sh = src[:, 2] - src[:, 0]
    sw = src[:, 3] - src[:, 1]
    cy = src[:, 0] + 0.5 * sh
    cx = src[:, 1] + 0.5 * sw
    dy, dx, dh, dw = loc[:, 0], loc[:, 1], loc[:, 2], loc[:, 3]
    ncy = dy * sh + cy
    ncx = dx * sw + cx
    nh = jnp.exp(dh) * sh
    nw = jnp.exp(dw) * sw
    return jnp.stack([ncy - 0.5 * nh, ncx - 0.5 * nw, ncy + 0.5 * nh, ncx + 0.5 * nw], axis=1)


def _conv2d(x, w, b, pad):
    out = jax.lax.conv_general_dilated(x, w, window_strides=(1, 1), padding=pad,
                                       dimension_numbers=('NCHW', 'OIHW', 'NCHW'))
    return out + b[None, :, None, None]


def _nms_body(y1_ref, x1_ref, y2_ref, x2_ref, alive_ref, keep_ref):
    y1 = y1_ref[...]
    x1 = x1_ref[...]
    y2 = y2_ref[...]
    x2 = x2_ref[...]
    area = (y2 - y1) * (x2 - x1)
    r = jax.lax.broadcasted_iota(jnp.int32, (_ROWS, _COLS), 0)
    c = jax.lax.broadcasted_iota(jnp.int32, (_ROWS, _COLS), 1)
    idx = r * _COLS + c
    big = jnp.int32(_PAD)
    alive0 = alive_ref[...] > jnp.float32(0.5)
    midx0 = jnp.where(alive0, idx, big)
    m0 = jnp.min(midx0)
    neg = jnp.float32(-1e30)

    def cond(carry):
        count, m, _, _ = carry
        return (count < _N_OUT) & (m < big)

    def body(carry):
        count, m, midx, keep = carry
        eq = idx == m
        y1k = jnp.max(jnp.where(eq, y1, neg))
        x1k = jnp.max(jnp.where(eq, x1, neg))
        y2k = jnp.max(jnp.where(eq, y2, neg))
        x2k = jnp.max(jnp.where(eq, x2, neg))
        areak = (y2k - y1k) * (x2k - x1k)
        iy1 = jnp.maximum(y1k, y1)
        ix1 = jnp.maximum(x1k, x1)
        iy2 = jnp.minimum(y2k, y2)
        ix2 = jnp.minimum(x2k, x2)
        inter = jnp.maximum(iy2 - iy1, jnp.float32(0.0)) * jnp.maximum(ix2 - ix1, jnp.float32(0.0))
        iou = inter / (areak + area - inter + jnp.float32(1e-9))
        supp = iou > jnp.float32(_THRESH)
        midx = jnp.where(supp | eq, big, midx)
        keep = jnp.where(eq, jnp.float32(1.0), keep)
        m2 = jnp.min(midx)
        return count + 1, m2, midx, keep

    keep0 = jnp.zeros((_ROWS, _COLS), dtype=jnp.float32)
    _, _, _, keep = jax.lax.while_loop(cond, body, (jnp.int32(0), m0, midx0, keep0))
    keep_ref[...] = keep


def _nms_keep_mask(rois_sorted):
    pad = jnp.zeros((_PAD - _N_IN, 4), dtype=jnp.float32)
    rp = jnp.concatenate([rois_sorted, pad], axis=0)
    y1 = rp[:, 0].reshape(_ROWS, _COLS)
    x1 = rp[:, 1].reshape(_ROWS, _COLS)
    y2 = rp[:, 2].reshape(_ROWS, _COLS)
    x2 = rp[:, 3].reshape(_ROWS, _COLS)
    alive0 = (jnp.arange(_PAD, dtype=jnp.int32) < _N_IN).astype(jnp.float32).reshape(_ROWS, _COLS)
    keep = pl.pallas_call(
        _nms_body,
        out_shape=jax.ShapeDtypeStruct((_ROWS, _COLS), jnp.float32),
    )(y1, x1, y2, x2, alive0)
    return keep.reshape(-1)[:_N_IN] > jnp.float32(0.5)


def kernel(x, conv_w, conv_b, score_w, score_b, loc_w, loc_b, img_size):
    feat_stride = 16
    anchors = _anchor_base(16, (0.5, 1, 2), (8, 16, 32))
    batch_size = x.shape[0]
    shifted = _shifted_anchors(anchors, feat_stride, x.shape[2], x.shape[3])
    h = jax.nn.relu(_conv2d(x, conv_w, conv_b, [(1, 1), (1, 1)]))
    rpn_locs = _conv2d(h, loc_w, loc_b, [(0, 0), (0, 0)])
    rpn_locs = jnp.transpose(rpn_locs, (0, 2, 3, 1)).reshape(batch_size, -1, 4)
    rpn_scores = _conv2d(h, score_w, score_b, [(0, 0), (0, 0)])
    rpn_scores = jnp.transpose(rpn_scores, (0, 2, 3, 1)).reshape(batch_size, -1, 2)
    rpn_fg_scores = jax.nn.softmax(rpn_scores, axis=2)[:, :, 1]

    locs = rpn_locs[0]
    scores = rpn_fg_scores[0]
    rois = _loc2bbox(shifted, locs)
    max_y = jnp.asarray(img_size[0], dtype=jnp.float32)
    max_x = jnp.asarray(img_size[1], dtype=jnp.float32)
    rois = rois.at[:, 0::2].set(jnp.clip(rois[:, 0::2], 0.0, max_y))
    rois = rois.at[:, 1::2].set(jnp.clip(rois[:, 1::2], 0.0, max_x))
    roi_h = rois[:, 2] - rois[:, 0]
    roi_w = rois[:, 3] - rois[:, 1]
    valid = (roi_h >= 16) & (roi_w >= 16)
    scores = jnp.where(valid, scores, -jnp.inf)
    order = jnp.argsort(-scores)[:_N_IN]
    rois_s = rois[order]

    keep_mask = _nms_keep_mask(rois_s)
    kept = jnp.nonzero(keep_mask, size=_N_OUT, fill_value=0)[0]
    rois_out = rois_s[kept]
    roi_indices = jnp.zeros((_N_OUT,), dtype=jnp.int32)
    return (rpn_scores, rpn_locs, rois_out, roi_indices, shifted)


# speedup vs baseline: 84.9501x; 1.0435x over previous
import jax
import jax.numpy as jnp
import numpy as np
from jax.experimental import pallas as pl

_N_IN = 12000
_N_OUT = 2000
_ROWS = 8
_COLS = _N_IN // _ROWS
_THRESH = 0.7


def _anchor_base(base_size=16, ratios=(0.5, 1, 2), scales=(8, 16, 32)):
    py = base_size / 2.0
    px = base_size / 2.0
    anchors = []
    for r in ratios:
        for s in scales:
            h = base_size * s * np.sqrt(r)
            w = base_size * s * np.sqrt(1.0 / r)
            anchors.append([py - h / 2.0, px - w / 2.0, py + h / 2.0, px + w / 2.0])
    return jnp.asarray(np.array(anchors, dtype=np.float32))


def _shifted_anchors(anchors, feat_stride, hh, ww):
    shift_y = jnp.arange(0, hh * feat_stride, feat_stride, dtype=jnp.float32)
    shift_x = jnp.arange(0, ww * feat_stride, feat_stride, dtype=jnp.float32)
    sy, sx = jnp.meshgrid(shift_y, shift_x, indexing='ij')
    shift = jnp.stack([sy.ravel(), sx.ravel(), sy.ravel(), sx.ravel()], axis=1)
    A = anchors.shape[0]
    K = shift.shape[0]
    all_anchors = anchors.reshape(1, A, 4) + shift.reshape(K, 1, 4)
    return all_anchors.reshape(K * A, 4)


def _loc2bbox(src, loc):
    sh = src[:, 2] - src[:, 0]
    sw = src[:, 3] - src[:, 1]
    cy = src[:, 0] + 0.5 * sh
    cx = src[:, 1] + 0.5 * sw
    dy, dx, dh, dw = loc[:, 0], loc[:, 1], loc[:, 2], loc[:, 3]
    ncy = dy * sh + cy
    ncx = dx * sw + cx
    nh = jnp.exp(dh) * sh
    nw = jnp.exp(dw) * sw
    return jnp.stack([ncy - 0.5 * nh, ncx - 0.5 * nw, ncy + 0.5 * nh, ncx + 0.5 * nw], axis=1)


def _conv2d(x, w, b, pad):
    out = jax.lax.conv_general_dilated(x, w, window_strides=(1, 1), padding=pad,
                                       dimension_numbers=('NCHW', 'OIHW', 'NCHW'))
    return out + b[None, :, None, None]


def _fold(v):
    rows = [v[i * _COLS:(i + 1) * _COLS].reshape(1, _COLS) for i in range(_ROWS)]
    return jnp.concatenate(rows, axis=0)


def _nms_body(y1_ref, x1_ref, y2_ref, x2_ref, keep_ref):
    y1 = _fold(y1_ref[...])
    x1 = _fold(x1_ref[...])
    y2 = _fold(y2_ref[...])
    x2 = _fold(x2_ref[...])
    area = (y2 - y1) * (x2 - x1)
    r = jax.lax.broadcasted_iota(jnp.int32, (_ROWS, _COLS), 0)
    c = jax.lax.broadcasted_iota(jnp.int32, (_ROWS, _COLS), 1)
    idx = r * _COLS + c
    big = jnp.int32(_N_IN)
    neg = jnp.float32(-1e30)

    def cond(carry):
        count, m, _, _ = carry
        return (count < _N_OUT) & (m < big)

    def body(carry):
        count, m, midx, keep = carry
        eq = idx == m
        y1k = jnp.max(jnp.where(eq, y1, neg))
        x1k = jnp.max(jnp.where(eq, x1, neg))
        y2k = jnp.max(jnp.where(eq, y2, neg))
        x2k = jnp.max(jnp.where(eq, x2, neg))
        areak = (y2k - y1k) * (x2k - x1k)
        iy1 = jnp.maximum(y1k, y1)
        ix1 = jnp.maximum(x1k, x1)
        iy2 = jnp.minimum(y2k, y2)
        ix2 = jnp.minimum(x2k, x2)
        inter = jnp.maximum(iy2 - iy1, jnp.float32(0.0)) * jnp.maximum(ix2 - ix1, jnp.float32(0.0))
        iou = inter / (areak + area - inter + jnp.float32(1e-9))
        supp = iou > jnp.float32(_THRESH)
        midx = jnp.where(supp | eq, big, midx)
        keep = jnp.where(eq, jnp.float32(1.0), keep)
        m2 = jnp.min(midx)
        return count + 1, m2, midx, keep

    keep0 = jnp.zeros((_ROWS, _COLS), dtype=jnp.float32)
    _, _, _, keep = jax.lax.while_loop(cond, body, (jnp.int32(0), jnp.int32(0), idx, keep0))
    keep_ref[...] = keep


def _nms_call(boxes):
    y1, x1, y2, x2 = boxes[:, 0], boxes[:, 1], boxes[:, 2], boxes[:, 3]
    keep = pl.pallas_call(
        _nms_body,
        out_shape=jax.ShapeDtypeStruct((_ROWS, _COLS), jnp.float32),
    )(y1, x1, y2, x2)
    return keep.reshape(-1) > jnp.float32(0.5)


def _nms_keep_mask(boxes):
    return jax.lax.cond(jnp.sum(boxes) > 0, _nms_call, _nms_call, boxes)


def kernel(x, conv_w, conv_b, score_w, score_b, loc_w, loc_b, img_size):
    feat_stride = 16
    anchors = _anchor_base(16, (0.5, 1, 2), (8, 16, 32))
    batch_size = x.shape[0]
    shifted = _shifted_anchors(anchors, feat_stride, x.shape[2], x.shape[3])
    h = jax.nn.relu(_conv2d(x, conv_w, conv_b, [(1, 1), (1, 1)]))
    rpn_locs = _conv2d(h, loc_w, loc_b, [(0, 0), (0, 0)])
    rpn_locs = jnp.transpose(rpn_locs, (0, 2, 3, 1)).reshape(batch_size, -1, 4)
    rpn_scores = _conv2d(h, score_w, score_b, [(0, 0), (0, 0)])
    rpn_scores = jnp.transpose(rpn_scores, (0, 2, 3, 1)).reshape(batch_size, -1, 2)
    rpn_fg_scores = jax.nn.softmax(rpn_scores, axis=2)[:, :, 1]

    locs = rpn_locs[0]
    scores = rpn_fg_scores[0]
    rois = _loc2bbox(shifted, locs)
    max_y = jnp.asarray(img_size[0], dtype=jnp.float32)
    max_x = jnp.asarray(img_size[1], dtype=jnp.float32)
    rois = rois.at[:, 0::2].set(jnp.clip(rois[:, 0::2], 0.0, max_y))
    rois = rois.at[:, 1::2].set(jnp.clip(rois[:, 1::2], 0.0, max_x))
    roi_h = rois[:, 2] - rois[:, 0]
    roi_w = rois[:, 3] - rois[:, 1]
    valid = (roi_h >= 16) & (roi_w >= 16)
    scores = jnp.where(valid, scores, -jnp.inf)
    order = jnp.argsort(-scores)[:_N_IN]
    rois_s = rois[order]

    keep_mask = _nms_keep_mask(rois_s)
    kept = jnp.nonzero(keep_mask, size=_N_OUT, fill_value=0)[0]
    rois_out = rois_s[kept]
    roi_indices = jnp.zeros((_N_OUT,), dtype=jnp.int32)
    return (rpn_scores, rpn_locs, rois_out, roi_indices, shifted)


# speedup vs baseline: 85.1943x; 1.0029x over previous
import jax
import jax.numpy as jnp
import numpy as np
from jax.experimental import pallas as pl

_N_IN = 12000
_N_OUT = 2000
_ROWS = 8
_COLS = _N_IN // _ROWS
_THRESH = 0.7


def _anchor_base(base_size=16, ratios=(0.5, 1, 2), scales=(8, 16, 32)):
    py = base_size / 2.0
    px = base_size / 2.0
    anchors = []
    for r in ratios:
        for s in scales:
            h = base_size * s * np.sqrt(r)
            w = base_size * s * np.sqrt(1.0 / r)
            anchors.append([py - h / 2.0, px - w / 2.0, py + h / 2.0, px + w / 2.0])
    return jnp.asarray(np.array(anchors, dtype=np.float32))


def _shifted_anchors(anchors, feat_stride, hh, ww):
    shift_y = jnp.arange(0, hh * feat_stride, feat_stride, dtype=jnp.float32)
    shift_x = jnp.arange(0, ww * feat_stride, feat_stride, dtype=jnp.float32)
    sy, sx = jnp.meshgrid(shift_y, shift_x, indexing='ij')
    shift = jnp.stack([sy.ravel(), sx.ravel(), sy.ravel(), sx.ravel()], axis=1)
    A = anchors.shape[0]
    K = shift.shape[0]
    all_anchors = anchors.reshape(1, A, 4) + shift.reshape(K, 1, 4)
    return all_anchors.reshape(K * A, 4)


def _loc2bbox(src, loc):
    sh = src[:, 2] - src[:, 0]
    sw = src[:, 3] - src[:, 1]
    cy = src[:, 0] + 0.5 * sh
    cx = src[:, 1] + 0.5 * sw
    dy, dx, dh, dw = loc[:, 0], loc[:, 1], loc[:, 2], loc[:, 3]
    ncy = dy * sh + cy
    ncx = dx * sw + cx
    nh = jnp.exp(dh) * sh
    nw = jnp.exp(dw) * sw
    return jnp.stack([ncy - 0.5 * nh, ncx - 0.5 * nw, ncy + 0.5 * nh, ncx + 0.5 * nw], axis=1)


def _conv2d(x, w, b, pad):
    out = jax.lax.conv_general_dilated(x, w, window_strides=(1, 1), padding=pad,
                                       dimension_numbers=('NCHW', 'OIHW', 'NCHW'))
    return out + b[None, :, None, None]


def _fold(v):
    rows = [v[i * _COLS:(i + 1) * _COLS].reshape(1, _COLS) for i in range(_ROWS)]
    return jnp.concatenate(rows, axis=0)


def _nms_body(y1_ref, x1_ref, y2_ref, x2_ref, keep_ref):
    y1 = _fold(y1_ref[...])
    x1 = _fold(x1_ref[...])
    y2 = _fold(y2_ref[...])
    x2 = _fold(x2_ref[...])
    area = (y2 - y1) * (x2 - x1)
    r = jax.lax.broadcasted_iota(jnp.int32, (_ROWS, _COLS), 0)
    c = jax.lax.broadcasted_iota(jnp.int32, (_ROWS, _COLS), 1)
    idx = r * _COLS + c
    big = jnp.int32(_N_IN)
    neg = jnp.float32(-1e30)

    def cond(carry):
        count, m, _, _ = carry
        return (count < _N_OUT) & (m < big)

    def body(carry):
        count, m, midx, keep = carry
        eq = idx == m
        y1k = jnp.max(jnp.where(eq, y1, neg))
        x1k = jnp.max(jnp.where(eq, x1, neg))
        y2k = jnp.max(jnp.where(eq, y2, neg))
        x2k = jnp.max(jnp.where(eq, x2, neg))
        areak = (y2k - y1k) * (x2k - x1k)
        iy1 = jnp.maximum(y1k, y1)
        ix1 = jnp.maximum(x1k, x1)
        iy2 = jnp.minimum(y2k, y2)
        ix2 = jnp.minimum(x2k, x2)
        inter = jnp.maximum(iy2 - iy1, jnp.float32(0.0)) * jnp.maximum(ix2 - ix1, jnp.float32(0.0))
        iou = inter / (areak + area - inter + jnp.float32(1e-9))
        supp = iou > jnp.float32(_THRESH)
        midx = jnp.where(supp | eq, big, midx)
        keep = jnp.where(eq, jnp.float32(1.0), keep)
        m2 = jnp.min(midx)
        return count + 1, m2, midx, keep

    keep0 = jnp.zeros((_ROWS, _COLS), dtype=jnp.float32)
    _, _, _, keep = jax.lax.while_loop(cond, body, (jnp.int32(0), jnp.int32(0), idx, keep0))
    keep_ref[...] = keep


def _nms_call(boxes):
    y1, x1, y2, x2 = boxes[:, 0], boxes[:, 1], boxes[:, 2], boxes[:, 3]
    keep = pl.pallas_call(
        _nms_body,
        out_shape=jax.ShapeDtypeStruct((_ROWS, _COLS), jnp.float32),
    )(y1, x1, y2, x2)
    return keep.reshape(-1) > jnp.float32(0.5)


def _nms_keep_mask(boxes):
    return jax.lax.cond(jnp.sum(boxes) > 0, _nms_call, _nms_call, boxes)


def kernel(x, conv_w, conv_b, score_w, score_b, loc_w, loc_b, img_size):
    feat_stride = 16
    anchors = _anchor_base(16, (0.5, 1, 2), (8, 16, 32))
    batch_size = x.shape[0]
    shifted = _shifted_anchors(anchors, feat_stride, x.shape[2], x.shape[3])
    h = jax.nn.relu(_conv2d(x, conv_w, conv_b, [(1, 1), (1, 1)]))
    rpn_locs = _conv2d(h, loc_w, loc_b, [(0, 0), (0, 0)])
    rpn_locs = jnp.transpose(rpn_locs, (0, 2, 3, 1)).reshape(batch_size, -1, 4)
    rpn_scores = _conv2d(h, score_w, score_b, [(0, 0), (0, 0)])
    rpn_scores = jnp.transpose(rpn_scores, (0, 2, 3, 1)).reshape(batch_size, -1, 2)
    rpn_fg_scores = jax.nn.softmax(rpn_scores, axis=2)[:, :, 1]

    locs = rpn_locs[0]
    scores = rpn_fg_scores[0]
    rois = _loc2bbox(shifted, locs)
    max_y = jnp.asarray(img_size[0], dtype=jnp.float32)
    max_x = jnp.asarray(img_size[1], dtype=jnp.float32)
    rois = rois.at[:, 0::2].set(jnp.clip(rois[:, 0::2], 0.0, max_y))
    rois = rois.at[:, 1::2].set(jnp.clip(rois[:, 1::2], 0.0, max_x))
    roi_h = rois[:, 2] - rois[:, 0]
    roi_w = rois[:, 3] - rois[:, 1]
    valid = (roi_h >= 16) & (roi_w >= 16)
    scores = jnp.where(valid, scores, -jnp.inf)
    order = jax.lax.top_k(scores, _N_IN)[1]
    rois_s = rois[order]

    keep_mask = _nms_keep_mask(rois_s)
    kept = jnp.nonzero(keep_mask, size=_N_OUT, fill_value=0)[0]
    rois_out = rois_s[kept]
    roi_indices = jnp.zeros((_N_OUT,), dtype=jnp.int32)
    return (rpn_scores, rpn_locs, rois_out, roi_indices, shifted)


# speedup vs baseline: 89.0421x; 1.0452x over previous
import jax
import jax.numpy as jnp
import numpy as np
from jax.experimental import pallas as pl

_N_IN = 12000
_N_OUT = 2000
_ROWS = 8
_COLS = _N_IN // _ROWS
_THRESH = 0.7


def _anchor_base(base_size=16, ratios=(0.5, 1, 2), scales=(8, 16, 32)):
    py = base_size / 2.0
    px = base_size / 2.0
    anchors = []
    for r in ratios:
        for s in scales:
            h = base_size * s * np.sqrt(r)
            w = base_size * s * np.sqrt(1.0 / r)
            anchors.append([py - h / 2.0, px - w / 2.0, py + h / 2.0, px + w / 2.0])
    return jnp.asarray(np.array(anchors, dtype=np.float32))


def _shifted_anchors(anchors, feat_stride, hh, ww):
    shift_y = jnp.arange(0, hh * feat_stride, feat_stride, dtype=jnp.float32)
    shift_x = jnp.arange(0, ww * feat_stride, feat_stride, dtype=jnp.float32)
    sy, sx = jnp.meshgrid(shift_y, shift_x, indexing='ij')
    shift = jnp.stack([sy.ravel(), sx.ravel(), sy.ravel(), sx.ravel()], axis=1)
    A = anchors.shape[0]
    K = shift.shape[0]
    all_anchors = anchors.reshape(1, A, 4) + shift.reshape(K, 1, 4)
    return all_anchors.reshape(K * A, 4)


def _loc2bbox(src, loc):
    sh = src[:, 2] - src[:, 0]
    sw = src[:, 3] - src[:, 1]
    cy = src[:, 0] + 0.5 * sh
    cx = src[:, 1] + 0.5 * sw
    dy, dx, dh, dw = loc[:, 0], loc[:, 1], loc[:, 2], loc[:, 3]
    ncy = dy * sh + cy
    ncx = dx * sw + cx
    nh = jnp.exp(dh) * sh
    nw = jnp.exp(dw) * sw
    return jnp.stack([ncy - 0.5 * nh, ncx - 0.5 * nw, ncy + 0.5 * nh, ncx + 0.5 * nw], axis=1)


def _conv2d(x, w, b, pad):
    out = jax.lax.conv_general_dilated(x, w, window_strides=(1, 1), padding=pad,
                                       dimension_numbers=('NCHW', 'OIHW', 'NCHW'))
    return out + b[None, :, None, None]


def _fold(v):
    rows = [v[i * _COLS:(i + 1) * _COLS].reshape(1, _COLS) for i in range(_ROWS)]
    return jnp.concatenate(rows, axis=0)


def _nms_body(y1_ref, x1_ref, y2_ref, x2_ref, boxes_ref, keep_ref):
    y1 = _fold(y1_ref[...])
    x1 = _fold(x1_ref[...])
    y2 = _fold(y2_ref[...])
    x2 = _fold(x2_ref[...])
    area = (y2 - y1) * (x2 - x1)
    r = jax.lax.broadcasted_iota(jnp.int32, (_ROWS, _COLS), 0)
    c = jax.lax.broadcasted_iota(jnp.int32, (_ROWS, _COLS), 1)
    idx = r * _COLS + c
    big = jnp.int32(_N_IN)

    def cond(carry):
        count, m, _, _ = carry
        return (count < _N_OUT) & (m < big)

    def body(carry):
        count, m, midx, keep = carry
        eq = idx == m
        bx = boxes_ref[pl.ds(m, 1), :]
        y1k = bx[0, 0]
        x1k = bx[0, 1]
        y2k = bx[0, 2]
        x2k = bx[0, 3]
        areak = (y2k - y1k) * (x2k - x1k)
        iy1 = jnp.maximum(y1k, y1)
        ix1 = jnp.maximum(x1k, x1)
        iy2 = jnp.minimum(y2k, y2)
        ix2 = jnp.minimum(x2k, x2)
        inter = jnp.maximum(iy2 - iy1, jnp.float32(0.0)) * jnp.maximum(ix2 - ix1, jnp.float32(0.0))
        iou = inter / (areak + area - inter + jnp.float32(1e-9))
        supp = iou > jnp.float32(_THRESH)
        midx = jnp.where(supp | eq, big, midx)
        keep = jnp.where(eq, jnp.float32(1.0), keep)
        m2 = jnp.min(midx)
        return count + 1, m2, midx, keep

    keep0 = jnp.zeros((_ROWS, _COLS), dtype=jnp.float32)
    _, _, _, keep = jax.lax.while_loop(cond, body, (jnp.int32(0), jnp.int32(0), idx, keep0))
    keep_ref[...] = keep


def _nms_call(boxes):
    y1, x1, y2, x2 = boxes[:, 0], boxes[:, 1], boxes[:, 2], boxes[:, 3]
    keep = pl.pallas_call(
        _nms_body,
        out_shape=jax.ShapeDtypeStruct((_ROWS, _COLS), jnp.float32),
    )(y1, x1, y2, x2, boxes)
    return keep.reshape(-1) > jnp.float32(0.5)


def _nms_keep_mask(boxes):
    return jax.lax.cond(jnp.sum(boxes) > 0, _nms_call, _nms_call, boxes)


def kernel(x, conv_w, conv_b, score_w, score_b, loc_w, loc_b, img_size):
    feat_stride = 16
    anchors = _anchor_base(16, (0.5, 1, 2), (8, 16, 32))
    batch_size = x.shape[0]
    shifted = _shifted_anchors(anchors, feat_stride, x.shape[2], x.shape[3])
    h = jax.nn.relu(_conv2d(x, conv_w, conv_b, [(1, 1), (1, 1)]))
    rpn_locs = _conv2d(h, loc_w, loc_b, [(0, 0), (0, 0)])
    rpn_locs = jnp.transpose(rpn_locs, (0, 2, 3, 1)).reshape(batch_size, -1, 4)
    rpn_scores = _conv2d(h, score_w, score_b, [(0, 0), (0, 0)])
    rpn_scores = jnp.transpose(rpn_scores, (0, 2, 3, 1)).reshape(batch_size, -1, 2)
    rpn_fg_scores = jax.nn.softmax(rpn_scores, axis=2)[:, :, 1]

    locs = rpn_locs[0]
    scores = rpn_fg_scores[0]
    rois = _loc2bbox(shifted, locs)
    max_y = jnp.asarray(img_size[0], dtype=jnp.float32)
    max_x = jnp.asarray(img_size[1], dtype=jnp.float32)
    rois = rois.at[:, 0::2].set(jnp.clip(rois[:, 0::2], 0.0, max_y))
    rois = rois.at[:, 1::2].set(jnp.clip(rois[:, 1::2], 0.0, max_x))
    roi_h = rois[:, 2] - rois[:, 0]
    roi_w = rois[:, 3] - rois[:, 1]
    valid = (roi_h >= 16) & (roi_w >= 16)
    scores = jnp.where(valid, scores, -jnp.inf)
    order = jax.lax.top_k(scores, _N_IN)[1]
    rois_s = rois[order]

    keep_mask = _nms_keep_mask(rois_s)
    kept = jnp.nonzero(keep_mask, size=_N_OUT, fill_value=0)[0]
    rois_out = rois_s[kept]
    roi_indices = jnp.zeros((_N_OUT,), dtype=jnp.int32)
    return (rpn_scores, rpn_locs, rois_out, roi_indices, shifted)
